# baseline (speedup 1.0000x reference)
import functools

import jax
import jax.numpy as jnp
from jax import lax
from jax.experimental import pallas as pl
from jax.experimental.pallas import tpu as pltpu
from jax.experimental.pallas import tpu_sc as plsc

_NUM_PT = 32
_NUM_T = 100000
_BATCH = 1024
_SEQ = 200
_SEQ_PAD = 208
_NB = 32
_NCHUNK = _BATCH // _NB
_NVEC = _SEQ_PAD // 16

_mesh = plsc.VectorSubcoreMesh(core_axis_name="c", subcore_axis_name="s")


@functools.partial(
    pl.kernel,
    mesh=_mesh,
    compiler_params=pltpu.CompilerParams(use_tc_tiling_on_sc=False,
                                          needs_layout_passes=False),
    out_type=jax.ShapeDtypeStruct((_BATCH, _NUM_PT, _SEQ), jnp.float32),
    scratch_types=[
        pltpu.VMEM((_NUM_T,), jnp.float32),
        pltpu.VMEM((2, _NB, _SEQ_PAD), jnp.int32),
        pltpu.VMEM((2, _NB, _SEQ_PAD), jnp.float32),
        pltpu.SemaphoreType.DMA,
        pltpu.SemaphoreType.DMA,
        pltpu.SemaphoreType.DMA,
        pltpu.SemaphoreType.DMA,
    ],
)
def _sc_lookup(sent_hbm, rules_hbm, out_hbm, row_v, idx_v, outbuf_v,
               sem_in0, sem_in1, sem_out0, sem_out1):
    wid = lax.axis_index("s") * 2 + lax.axis_index("c")
    sem_in = (sem_in0, sem_in1)
    sem_out = (sem_out0, sem_out1)

    def in_copy(ci, b):
        return pltpu.make_async_copy(sent_hbm.at[pl.ds(ci * _NB, _NB)],
                                     idx_v.at[b, :, pl.ds(0, _SEQ)], sem_in[b])

    def out_copy(ci, b):
        return pltpu.make_async_copy(outbuf_v.at[b, :, pl.ds(0, _SEQ)],
                                     out_hbm.at[pl.ds(ci * _NB, _NB), wid],
                                     sem_out[b])

    @plsc.parallel_loop(0, _NB)
    def _zero_pad(s):
        for bb in range(2):
            idx_v[bb, s, pl.ds(_SEQ_PAD - 16, 16)] = jnp.zeros(16, jnp.int32)

    in_copy(0, 0).start()
    in_copy(1, 1).start()
    pltpu.sync_copy(rules_hbm.at[wid], row_v)

    def pair_body(p, _):
        for b in range(2):
            ci = p * 2 + b
            in_copy(ci, b).wait()

            @pl.when(p > 0)
            def _wait_out():
                out_copy(ci - 2, b).wait()

            @plsc.parallel_loop(0, _NB, unroll=2)
            def sent_body(s):
                for j in range(_NVEC):
                    idx = idx_v[b, s, pl.ds(j * 16, 16)]
                    outbuf_v[b, s, pl.ds(j * 16, 16)] = (
                        plsc.load_gather(row_v, [idx]))
            out_copy(ci, b).start()

            @pl.when(ci + 2 < _NCHUNK)
            def _prefetch():
                in_copy(ci + 2, b).start()
        return 0

    lax.fori_loop(0, _NCHUNK // 2, pair_body, 0)
    for b in range(2):
        out_copy(_NCHUNK - 2 + b, b).wait()


def kernel(sentences, rules):
    return _sc_lookup(sentences.astype(jnp.int32), rules)


# speedup vs baseline: 1.9127x; 1.9127x over previous
import functools

import jax
import jax.numpy as jnp
from jax import lax
from jax.experimental import pallas as pl
from jax.experimental.pallas import tpu as pltpu
from jax.experimental.pallas import tpu_sc as plsc

_NUM_PT = 32
_NUM_T = 100000
_BATCH = 1024
_SEQ = 200
_TOK = _BATCH * _SEQ
_CHUNK = 4096
_NCHUNK = _TOK // _CHUNK
_NVEC = _CHUNK // 16

_mesh = plsc.VectorSubcoreMesh(core_axis_name="c", subcore_axis_name="s")


@functools.partial(
    pl.kernel,
    mesh=_mesh,
    compiler_params=pltpu.CompilerParams(use_tc_tiling_on_sc=False,
                                         needs_layout_passes=False),
    out_type=jax.ShapeDtypeStruct((_NUM_PT, _TOK), jnp.float32),
    scratch_types=[
        pltpu.VMEM((_NUM_T,), jnp.float32),
        pltpu.VMEM((2, _CHUNK), jnp.int32),
        pltpu.VMEM((2, _CHUNK), jnp.float32),
        pltpu.SemaphoreType.DMA,
        pltpu.SemaphoreType.DMA,
        pltpu.SemaphoreType.DMA,
        pltpu.SemaphoreType.DMA,
    ],
)
def _sc_lookup(idx_hbm, rules_hbm, out_hbm, row_v, idx_v, out_v,
               sem_in0, sem_in1, sem_out0, sem_out1):
    wid = lax.axis_index("s") * 2 + lax.axis_index("c")
    sem_in = (sem_in0, sem_in1)
    sem_out = (sem_out0, sem_out1)

    def in_copy(ci, b):
        return pltpu.make_async_copy(idx_hbm.at[pl.ds(ci * _CHUNK, _CHUNK)],
                                     idx_v.at[b], sem_in[b])

    def out_copy(ci, b):
        return pltpu.make_async_copy(out_v.at[b],
                                     out_hbm.at[wid, pl.ds(ci * _CHUNK, _CHUNK)],
                                     sem_out[b])

    in_copy(0, 0).start()
    in_copy(1, 1).start()
    pltpu.sync_copy(rules_hbm.at[wid], row_v)

    def pair_body(p, _):
        for b in range(2):
            ci = p * 2 + b
            in_copy(ci, b).wait()

            @pl.when(p > 0)
            def _wait_out():
                out_copy(ci - 2, b).wait()

            @plsc.parallel_loop(0, _NVEC, unroll=4)
            def vec_body(v):
                idx = idx_v[b, pl.ds(v * 16, 16)]
                out_v[b, pl.ds(v * 16, 16)] = plsc.load_gather(row_v, [idx])

            out_copy(ci, b).start()

            @pl.when(ci + 2 < _NCHUNK)
            def _prefetch():
                in_copy(ci + 2, b).start()
        return 0

    lax.fori_loop(0, _NCHUNK // 2, pair_body, 0)
    for b in range(2):
        out_copy(_NCHUNK - 2 + b, b).wait()


def kernel(sentences, rules):
    idx5 = (sentences.astype(jnp.int32)
            .T.reshape(_SEQ // 8, 8, _BATCH // 128, 128)
            .transpose(0, 2, 1, 3)
            .reshape(_TOK))
    out5 = _sc_lookup(idx5, rules)
    return (out5
            .reshape(_NUM_PT, _SEQ // 8, _BATCH // 128, 8, 128)
            .transpose(2, 4, 0, 1, 3)
            .reshape(_BATCH, _NUM_PT, _SEQ))


# speedup vs baseline: 1.9150x; 1.0012x over previous
import functools

import jax
import jax.numpy as jnp
from jax import lax
from jax.experimental import pallas as pl
from jax.experimental.pallas import tpu as pltpu
from jax.experimental.pallas import tpu_sc as plsc

_NUM_PT = 32
_NUM_T = 100000
_BATCH = 1024
_SEQ = 200
_TOK = _BATCH * _SEQ
_CHUNK = 4096
_NCHUNK = _TOK // _CHUNK
_NVEC = _CHUNK // 16

_mesh = plsc.VectorSubcoreMesh(core_axis_name="c", subcore_axis_name="s")


@functools.partial(
    pl.kernel,
    mesh=_mesh,
    compiler_params=pltpu.CompilerParams(use_tc_tiling_on_sc=False,
                                         needs_layout_passes=False),
    out_type=jax.ShapeDtypeStruct((_NUM_PT, _TOK), jnp.float32),
    scratch_types=[
        pltpu.VMEM((_NUM_T,), jnp.float32),
        pltpu.VMEM((2, _CHUNK), jnp.int32),
        pltpu.VMEM((2, _CHUNK), jnp.float32),
        pltpu.SemaphoreType.DMA,
        pltpu.SemaphoreType.DMA,
        pltpu.SemaphoreType.DMA,
        pltpu.SemaphoreType.DMA,
    ],
)
def _sc_lookup(idx_hbm, rules_hbm, out_hbm, row_v, idx_v, out_v,
               sem_in0, sem_in1, sem_out0, sem_out1):
    wid = lax.axis_index("s") * 2 + lax.axis_index("c")
    sem_in = (sem_in0, sem_in1)
    sem_out = (sem_out0, sem_out1)

    def in_copy(ci, b):
        return pltpu.make_async_copy(idx_hbm.at[pl.ds(ci * _CHUNK, _CHUNK)],
                                     idx_v.at[b], sem_in[b])

    def out_copy(ci, b):
        return pltpu.make_async_copy(out_v.at[b],
                                     out_hbm.at[wid, pl.ds(ci * _CHUNK, _CHUNK)],
                                     sem_out[b])

    in_copy(0, 0).start()
    in_copy(1, 1).start()
    pltpu.sync_copy(rules_hbm.at[wid], row_v)

    def pair_body(p, _):
        for b in range(2):
            ci = p * 2 + b
            in_copy(ci, b).wait()

            @pl.when(p > 0)
            def _wait_out():
                out_copy(ci - 2, b).wait()

            @plsc.parallel_loop(0, _NVEC, unroll=8)
            def vec_body(v):
                idx = idx_v[b, pl.ds(v * 16, 16)]
                out_v[b, pl.ds(v * 16, 16)] = plsc.load_gather(row_v, [idx])

            out_copy(ci, b).start()

            @pl.when(ci + 2 < _NCHUNK)
            def _prefetch():
                in_copy(ci + 2, b).start()
        return 0

    lax.fori_loop(0, _NCHUNK // 2, pair_body, 0)
    for b in range(2):
        out_copy(_NCHUNK - 2 + b, b).wait()


def kernel(sentences, rules):
    idx5 = (sentences.astype(jnp.int32)
            .T.reshape(_SEQ // 8, 8, _BATCH // 128, 128)
            .transpose(0, 2, 1, 3)
            .reshape(_TOK))
    out5 = _sc_lookup(idx5, rules)
    return (out5
            .reshape(_NUM_PT, _SEQ // 8, _BATCH // 128, 8, 128)
            .transpose(2, 4, 0, 1, 3)
            .reshape(_BATCH, _NUM_PT, _SEQ))
